# baseline (device time: 33701 ns/iter reference)
import jax
import jax.numpy as jnp
from jax import lax
from jax.experimental import pallas as pl
from jax.experimental.pallas import tpu as pltpu

N_DEV = 4


def kernel(x, router_W, route_idx, expert_W):
    n_tok, d_model = x.shape
    e_local, _, d_out = expert_W.shape
    n_experts = router_W.shape[1]
    rows_per = n_tok // N_DEV

    def body(x_ref, rw_ref, idx_ref, ew_ref, out_ref,
             partial_ref, send_buf, recv_buf, send_sems, recv_sems):
        my_i = lax.axis_index("i")
        left = lax.rem(my_i + N_DEV - 1, N_DEV)
        right = lax.rem(my_i + 1, N_DEV)

        barrier_sem = pltpu.get_barrier_semaphore()
        for nbr in [left, right]:
            pl.semaphore_signal(
                barrier_sem, inc=1,
                device_id=(nbr,), device_id_type=pl.DeviceIdType.MESH,
            )
        pl.semaphore_wait(barrier_sem, 2)

        xv = x_ref[:, :]
        scores = jnp.dot(xv, rw_ref[:, :], preferred_element_type=jnp.float32)
        s_max = jnp.max(scores, axis=-1, keepdims=True)
        p = jnp.exp(scores - s_max)
        probs = p / jnp.sum(p, axis=-1, keepdims=True)
        e0 = idx_ref[:, 0:1]
        e1 = idx_ref[:, 1:2]
        iota = lax.broadcasted_iota(jnp.int32, (n_tok, n_experts), 1)
        top2 = jnp.logical_or(e0 == iota, e1 == iota).astype(jnp.float32)
        gs = jnp.sum(probs * top2, axis=-1, keepdims=True)
        w = probs * top2 / gs

        for j in range(e_local):
            ge = my_i * e_local + j
            wj = jnp.sum(
                w * (iota == ge).astype(jnp.float32), axis=-1, keepdims=True
            )
            pj = jnp.dot(
                xv * wj, ew_ref[j], preferred_element_type=jnp.float32
            )
            if j == 0:
                partial_ref[:, :] = pj
            else:
                partial_ref[:, :] = partial_ref[:, :] + pj

        chunk0 = lax.rem(my_i + N_DEV - 1, N_DEV)
        send_buf[0] = partial_ref[pl.ds(chunk0 * rows_per, rows_per), :]
        for h in range(N_DEV - 1):
            rdma = pltpu.make_async_remote_copy(
                src_ref=send_buf.at[h],
                dst_ref=recv_buf.at[h],
                send_sem=send_sems.at[h],
                recv_sem=recv_sems.at[h],
                device_id=(right,),
                device_id_type=pl.DeviceIdType.MESH,
            )
            rdma.start()
            rdma.wait()
            c = lax.rem(my_i + 2 * N_DEV - 2 - h, N_DEV)
            acc = recv_buf[h] + partial_ref[pl.ds(c * rows_per, rows_per), :]
            if h < N_DEV - 2:
                send_buf[h + 1] = acc
            else:
                out_ref[:, :] = acc

    return pl.pallas_call(
        body,
        out_shape=jax.ShapeDtypeStruct((rows_per, d_out), jnp.float32),
        in_specs=[pl.BlockSpec(memory_space=pltpu.VMEM)] * 4,
        out_specs=pl.BlockSpec(memory_space=pltpu.VMEM),
        scratch_shapes=[
            pltpu.VMEM((n_tok, d_out), jnp.float32),
            pltpu.VMEM((N_DEV - 1, rows_per, d_out), jnp.float32),
            pltpu.VMEM((N_DEV - 1, rows_per, d_out), jnp.float32),
            pltpu.SemaphoreType.DMA((N_DEV - 1,)),
            pltpu.SemaphoreType.DMA((N_DEV - 1,)),
        ],
        compiler_params=pltpu.CompilerParams(collective_id=0),
    )(x, router_W, route_idx, expert_W)


# device time: 24355 ns/iter; 1.3837x vs baseline; 1.3837x over previous
import jax
import jax.numpy as jnp
from jax import lax
from jax.experimental import pallas as pl
from jax.experimental.pallas import tpu as pltpu

N_DEV = 4


def kernel(x, router_W, route_idx, expert_W):
    n_tok, d_model = x.shape
    e_local, _, d_out = expert_W.shape
    n_experts = router_W.shape[1]
    rows_per = n_tok // N_DEV

    def body(x_ref, rw_ref, idx_ref, ew_ref, out_ref,
             xw_ref, send_buf, recv_buf, send_sems, recv_sems):
        my_i = lax.axis_index("i")
        left = lax.rem(my_i + N_DEV - 1, N_DEV)
        right = lax.rem(my_i + 1, N_DEV)
        diag = lax.rem(my_i + 2, N_DEV)

        barrier_sem = pltpu.get_barrier_semaphore()
        for nbr in [left, right, diag]:
            pl.semaphore_signal(
                barrier_sem, inc=1,
                device_id=(nbr,), device_id_type=pl.DeviceIdType.MESH,
            )
        pl.semaphore_wait(barrier_sem, N_DEV - 1)

        xv = x_ref[:, :]
        scores = jnp.dot(xv, rw_ref[:, :], preferred_element_type=jnp.float32)
        s_max = jnp.max(scores, axis=-1, keepdims=True)
        p = jnp.exp(scores - s_max)
        probs = p / jnp.sum(p, axis=-1, keepdims=True)
        e0 = idx_ref[:, 0:1]
        e1 = idx_ref[:, 1:2]
        iota = lax.broadcasted_iota(jnp.int32, (n_tok, n_experts), 1)
        top2 = jnp.logical_or(e0 == iota, e1 == iota).astype(jnp.float32)
        gs = jnp.sum(probs * top2, axis=-1, keepdims=True)
        w = probs * top2 / gs

        for j in range(e_local):
            ge = my_i * e_local + j
            wj = jnp.sum(
                w * (iota == ge).astype(jnp.float32), axis=-1, keepdims=True
            )
            xw_ref[:, j * d_model:(j + 1) * d_model] = xv * wj

        ew = ew_ref[:, :, :].reshape(e_local * d_model, d_out)

        rdmas = []
        for k, (tgt, slot) in enumerate([(diag, 1), (left, 2), (right, 0)]):
            chunk = xw_ref[pl.ds(tgt * rows_per, rows_per), :]
            send_buf[k] = jnp.dot(chunk, ew, preferred_element_type=jnp.float32)
            rdma = pltpu.make_async_remote_copy(
                src_ref=send_buf.at[k],
                dst_ref=recv_buf.at[slot],
                send_sem=send_sems.at[k],
                recv_sem=recv_sems.at[slot],
                device_id=(tgt,),
                device_id_type=pl.DeviceIdType.MESH,
            )
            rdma.start()
            rdmas.append(rdma)

        own = jnp.dot(
            xw_ref[pl.ds(my_i * rows_per, rows_per), :], ew,
            preferred_element_type=jnp.float32,
        )

        for rdma in rdmas:
            rdma.wait_send()
        for rdma in rdmas:
            rdma.wait_recv()
        out_ref[:, :] = own + recv_buf[0] + recv_buf[1] + recv_buf[2]

    return pl.pallas_call(
        body,
        out_shape=jax.ShapeDtypeStruct((rows_per, d_out), jnp.float32),
        in_specs=[pl.BlockSpec(memory_space=pltpu.VMEM)] * 4,
        out_specs=pl.BlockSpec(memory_space=pltpu.VMEM),
        scratch_shapes=[
            pltpu.VMEM((n_tok, e_local * d_model), jnp.float32),
            pltpu.VMEM((N_DEV - 1, rows_per, d_out), jnp.float32),
            pltpu.VMEM((N_DEV - 1, rows_per, d_out), jnp.float32),
            pltpu.SemaphoreType.DMA((N_DEV - 1,)),
            pltpu.SemaphoreType.DMA((N_DEV - 1,)),
        ],
        compiler_params=pltpu.CompilerParams(collective_id=0),
    )(x, router_W, route_idx, expert_W)


# device time: 17523 ns/iter; 1.9232x vs baseline; 1.3899x over previous
import jax
import jax.numpy as jnp
from jax import lax
from jax.experimental import pallas as pl
from jax.experimental.pallas import tpu as pltpu

N_DEV = 4


def kernel(x, router_W, route_idx, expert_W):
    n_tok, d_model = x.shape
    e_local, _, d_out = expert_W.shape
    n_experts = router_W.shape[1]
    rows_per = n_tok // N_DEV

    def body(x_ref, rw_ref, idx_ref, ew_ref, out_ref,
             xw_ref, send_buf, recv_buf, send_sems, recv_sems):
        my_i = lax.axis_index("i")
        left = lax.rem(my_i + N_DEV - 1, N_DEV)
        right = lax.rem(my_i + 1, N_DEV)
        diag = lax.rem(my_i + 2, N_DEV)

        barrier_sem = pltpu.get_barrier_semaphore()
        for nbr in [left, right, diag]:
            pl.semaphore_signal(
                barrier_sem, inc=1,
                device_id=(nbr,), device_id_type=pl.DeviceIdType.MESH,
            )

        xv = x_ref[:, :]
        scores = jnp.dot(xv, rw_ref[:, :], preferred_element_type=jnp.float32)
        s_max = jnp.max(scores, axis=-1, keepdims=True)
        p = jnp.exp(scores - s_max)
        probs = p / jnp.sum(p, axis=-1, keepdims=True)
        e0 = idx_ref[:, 0:1]
        e1 = idx_ref[:, 1:2]
        iota = lax.broadcasted_iota(jnp.int32, (n_tok, n_experts), 1)
        top2 = jnp.logical_or(e0 == iota, e1 == iota).astype(jnp.float32)
        gs = jnp.sum(probs * top2, axis=-1, keepdims=True)
        w = probs * top2 / gs

        for j in range(e_local):
            ge = my_i * e_local + j
            wj = jnp.sum(
                w * (iota == ge).astype(jnp.float32), axis=-1, keepdims=True
            )
            xw_ref[:, j * d_model:(j + 1) * d_model] = (
                (xv * wj).astype(jnp.bfloat16)
            )

        ew = ew_ref[:, :, :].reshape(e_local * d_model, d_out)
        ew = ew.astype(jnp.bfloat16)

        rdmas = []
        for k, (tgt, slot) in enumerate([(diag, 1), (left, 2), (right, 0)]):
            chunk = xw_ref[pl.ds(tgt * rows_per, rows_per), :]
            send_buf[k] = jnp.dot(
                chunk, ew, preferred_element_type=jnp.float32
            ).astype(jnp.bfloat16)
            if k == 0:
                pl.semaphore_wait(barrier_sem, N_DEV - 1)
            rdma = pltpu.make_async_remote_copy(
                src_ref=send_buf.at[k],
                dst_ref=recv_buf.at[slot],
                send_sem=send_sems.at[k],
                recv_sem=recv_sems.at[slot],
                device_id=(tgt,),
                device_id_type=pl.DeviceIdType.MESH,
            )
            rdma.start()
            rdmas.append(rdma)

        own = jnp.dot(
            xw_ref[pl.ds(my_i * rows_per, rows_per), :], ew,
            preferred_element_type=jnp.float32,
        )

        for rdma in rdmas:
            rdma.wait_send()
        for rdma in rdmas:
            rdma.wait_recv()
        out_ref[:, :] = (
            own
            + recv_buf[0].astype(jnp.float32)
            + recv_buf[1].astype(jnp.float32)
            + recv_buf[2].astype(jnp.float32)
        )

    return pl.pallas_call(
        body,
        out_shape=jax.ShapeDtypeStruct((rows_per, d_out), jnp.float32),
        in_specs=[pl.BlockSpec(memory_space=pltpu.VMEM)] * 4,
        out_specs=pl.BlockSpec(memory_space=pltpu.VMEM),
        scratch_shapes=[
            pltpu.VMEM((n_tok, e_local * d_model), jnp.bfloat16),
            pltpu.VMEM((N_DEV - 1, rows_per, d_out), jnp.bfloat16),
            pltpu.VMEM((N_DEV - 1, rows_per, d_out), jnp.bfloat16),
            pltpu.SemaphoreType.DMA((N_DEV - 1,)),
            pltpu.SemaphoreType.DMA((N_DEV - 1,)),
        ],
        compiler_params=pltpu.CompilerParams(collective_id=0),
    )(x, router_W, route_idx, expert_W)


# device time: 16894 ns/iter; 1.9949x vs baseline; 1.0372x over previous
import jax
import jax.numpy as jnp
from jax import lax
from jax.experimental import pallas as pl
from jax.experimental.pallas import tpu as pltpu

N_DEV = 4


def kernel(x, router_W, route_idx, expert_W):
    n_tok, d_model = x.shape
    e_local, _, d_out = expert_W.shape
    n_experts = router_W.shape[1]
    rows_per = n_tok // N_DEV

    def body(x_ref, rw_ref, idx_ref, ew_ref, out_ref,
             w_ref, send_buf, recv_buf, send_sems, recv_sems):
        my_i = lax.axis_index("i")
        left = lax.rem(my_i + N_DEV - 1, N_DEV)
        right = lax.rem(my_i + 1, N_DEV)
        diag = lax.rem(my_i + 2, N_DEV)

        barrier_sem = pltpu.get_barrier_semaphore()
        for nbr in [left, right, diag]:
            pl.semaphore_signal(
                barrier_sem, inc=1,
                device_id=(nbr,), device_id_type=pl.DeviceIdType.MESH,
            )

        xv = x_ref[:, :]
        scores = jnp.dot(xv, rw_ref[:, :], preferred_element_type=jnp.float32)
        s_max = jnp.max(scores, axis=-1, keepdims=True)
        p = jnp.exp(scores - s_max)
        probs = p / jnp.sum(p, axis=-1, keepdims=True)
        e0 = idx_ref[:, 0:1]
        e1 = idx_ref[:, 1:2]
        iota = lax.broadcasted_iota(jnp.int32, (n_tok, n_experts), 1)
        top2 = jnp.logical_or(e0 == iota, e1 == iota).astype(jnp.float32)
        gs = jnp.sum(probs * top2, axis=-1, keepdims=True)
        w_ref[:, :] = probs * top2 / gs

        ew = ew_ref[:, :, :].reshape(e_local * d_model, d_out)
        ew = ew.astype(jnp.bfloat16)

        def chunk_partial(c):
            xc = x_ref[pl.ds(c * rows_per, rows_per), :]
            wc = w_ref[pl.ds(c * rows_per, rows_per), :]
            iota_c = lax.broadcasted_iota(jnp.int32, (rows_per, n_experts), 1)
            pieces = []
            for j in range(e_local):
                ge = my_i * e_local + j
                wjc = jnp.sum(
                    wc * (iota_c == ge).astype(jnp.float32),
                    axis=-1, keepdims=True,
                )
                pieces.append((xc * wjc).astype(jnp.bfloat16))
            xwc = jnp.concatenate(pieces, axis=1)
            return jnp.dot(xwc, ew, preferred_element_type=jnp.float32)

        rdmas = []
        for k, (tgt, slot) in enumerate([(diag, 1), (left, 2), (right, 0)]):
            send_buf[k] = chunk_partial(tgt).astype(jnp.bfloat16)
            if k == 0:
                pl.semaphore_wait(barrier_sem, N_DEV - 1)
            rdma = pltpu.make_async_remote_copy(
                src_ref=send_buf.at[k],
                dst_ref=recv_buf.at[slot],
                send_sem=send_sems.at[k],
                recv_sem=recv_sems.at[slot],
                device_id=(tgt,),
                device_id_type=pl.DeviceIdType.MESH,
            )
            rdma.start()
            rdmas.append(rdma)

        acc = chunk_partial(my_i)
        for rdma, (_, slot) in zip(rdmas, [(diag, 1), (left, 2), (right, 0)]):
            rdma.wait_recv()
            acc = acc + recv_buf[slot].astype(jnp.float32)
        out_ref[:, :] = acc
        for rdma in rdmas:
            rdma.wait_send()

    return pl.pallas_call(
        body,
        out_shape=jax.ShapeDtypeStruct((rows_per, d_out), jnp.float32),
        in_specs=[pl.BlockSpec(memory_space=pltpu.VMEM)] * 4,
        out_specs=pl.BlockSpec(memory_space=pltpu.VMEM),
        scratch_shapes=[
            pltpu.VMEM((n_tok, n_experts), jnp.float32),
            pltpu.VMEM((N_DEV - 1, rows_per, d_out), jnp.bfloat16),
            pltpu.VMEM((N_DEV - 1, rows_per, d_out), jnp.bfloat16),
            pltpu.SemaphoreType.DMA((N_DEV - 1,)),
            pltpu.SemaphoreType.DMA((N_DEV - 1,)),
        ],
        compiler_params=pltpu.CompilerParams(collective_id=0),
    )(x, router_W, route_idx, expert_W)
